# Initial kernel scaffold; baseline (speedup 1.0000x reference)
import jax
import jax.numpy as jnp
from jax.experimental import pallas as pl


def kernel(hidden_states, Wg, W_gate_up, W_down, Ws_gate_up, Ws_down):
    raise NotImplementedError("write your pallas kernel here")


# speedup vs baseline: 1.3416x; 1.3416x over previous
import functools

import jax
import jax.numpy as jnp
from jax.experimental import pallas as pl
from jax.experimental.pallas import tpu as pltpu

E = 8
TOPK = 2
D = 2048
FF = 4096
T = 2048

TR = 256
NR = T // TR
TF = 512
NF = FF // TF


def _routing_body(x_ref, wg_ref, counts_ref, post_ref, fwt_ref):
    logits = jax.lax.dot_general(
        wg_ref[...], x_ref[...], (((0,), (1,)), ((), ())),
        preferred_element_type=jnp.float32)
    m = jnp.max(logits, axis=0, keepdims=True)
    p = jnp.exp(logits - m)
    p = p / jnp.sum(p, axis=0, keepdims=True)

    eidx = jax.lax.broadcasted_iota(jnp.int32, (E, T), 0)
    w1 = jnp.max(p, axis=0, keepdims=True)
    i1 = jnp.min(jnp.where(p == w1, eidx, E), axis=0, keepdims=True)
    p2 = jnp.where(eidx == i1, -jnp.inf, p)
    w2 = jnp.max(p2, axis=0, keepdims=True)
    i2 = jnp.min(jnp.where(p2 == w2, eidx, E), axis=0, keepdims=True)
    s = w1 + w2
    w1n = w1 / s
    w2n = w2 / s

    sel1 = (eidx == i1)
    sel2 = (eidx == i2)
    mask = (sel1 | sel2).astype(jnp.float32)
    fwt = jnp.where(sel1, w1n, 0.0) + jnp.where(sel2, w2n, 0.0)

    ti = jax.lax.broadcasted_iota(jnp.int32, (T, T), 0)
    tj = jax.lax.broadcasted_iota(jnp.int32, (T, T), 1)
    upper = (ti < tj).astype(jnp.float32)
    pos = jax.lax.dot_general(
        mask, upper, (((1,), (0,)), ((), ())),
        preferred_element_type=jnp.float32)

    post_ref[...] = jnp.where(mask > 0.0, pos, -1.0)
    fwt_ref[...] = fwt
    counts_ref[...] = jnp.sum(mask, axis=1, keepdims=True).astype(jnp.int32)


def _routing(x, Wg):
    return pl.pallas_call(
        _routing_body,
        out_shape=(
            jax.ShapeDtypeStruct((E, 1), jnp.int32),
            jax.ShapeDtypeStruct((E, T), jnp.float32),
            jax.ShapeDtypeStruct((E, T), jnp.float32),
        ),
        compiler_params=pltpu.CompilerParams(
            vmem_limit_bytes=100 * 1024 * 1024),
    )(x, Wg)


def _expert_body(counts_s, post_ref, fwt_ref, x_ref, wg_ref, wu_ref, wd_ref,
                 out_ref, acc_ref, xg_ref):
    e = pl.program_id(0)
    r = pl.program_id(1)
    f = pl.program_id(2)

    @pl.when((e == 0) & (r == 0) & (f == 0))
    def _init():
        out_ref[...] = jnp.zeros_like(out_ref)

    active = (r * TR) < counts_s[e]

    def _onehot():
        pos_row = post_ref[0]
        slot = ((r * TR) + jax.lax.broadcasted_iota(
            jnp.int32, (TR, 1), 0)).astype(jnp.float32)
        return (pos_row == slot).astype(jnp.float32)

    @pl.when(active & (f == 0))
    def _gather():
        g = _onehot()
        xg_ref[...] = jax.lax.dot_general(
            g, x_ref[...], (((1,), (0,)), ((), ())),
            preferred_element_type=jnp.float32)

    @pl.when(active)
    def _mlp():
        xg = xg_ref[...]
        gate = jax.lax.dot_general(
            xg, wg_ref[0], (((1,), (0,)), ((), ())),
            preferred_element_type=jnp.float32)
        up = jax.lax.dot_general(
            xg, wu_ref[0], (((1,), (0,)), ((), ())),
            preferred_element_type=jnp.float32)
        act = jax.nn.silu(gate) * up
        part = jax.lax.dot_general(
            act, wd_ref[0], (((1,), (0,)), ((), ())),
            preferred_element_type=jnp.float32)

        @pl.when(f == 0)
        def _():
            acc_ref[...] = part

        @pl.when(f > 0)
        def _():
            acc_ref[...] = acc_ref[...] + part

    @pl.when(active & (f == NF - 1))
    def _scatter():
        g = _onehot() * fwt_ref[0]
        out_ref[...] += jax.lax.dot_general(
            g, acc_ref[...], (((0,), (0,)), ((), ())),
            preferred_element_type=jnp.float32)


def _expert_block(counts, post, fwt, x, W_gate_up, W_down):
    def tab_idx(e, r, f, c):
        return (e, 0, 0)

    def wgate_idx(e, r, f, c):
        fe = jnp.where((r * TR) < c[e], f, NF - 1)
        return (e, 0, fe)

    def wup_idx(e, r, f, c):
        fe = jnp.where((r * TR) < c[e], f, NF - 1)
        return (e, 0, fe + NF)

    def wdown_idx(e, r, f, c):
        fe = jnp.where((r * TR) < c[e], f, NF - 1)
        return (e, fe, 0)

    grid_spec = pltpu.PrefetchScalarGridSpec(
        num_scalar_prefetch=1,
        grid=(E, NR, NF),
        in_specs=[
            pl.BlockSpec((1, 1, T), tab_idx),
            pl.BlockSpec((1, 1, T), tab_idx),
            pl.BlockSpec((T, D), lambda e, r, f, c: (0, 0)),
            pl.BlockSpec((1, D, TF), wgate_idx),
            pl.BlockSpec((1, D, TF), wup_idx),
            pl.BlockSpec((1, TF, D), wdown_idx),
        ],
        out_specs=pl.BlockSpec((T, D), lambda e, r, f, c: (0, 0)),
        scratch_shapes=[
            pltpu.VMEM((TR, D), jnp.float32),
            pltpu.VMEM((TR, D), jnp.float32),
        ],
    )
    return pl.pallas_call(
        _expert_body,
        grid_spec=grid_spec,
        out_shape=jax.ShapeDtypeStruct((T, D), jnp.float32),
        compiler_params=pltpu.CompilerParams(
            dimension_semantics=("arbitrary", "arbitrary", "arbitrary"),
            vmem_limit_bytes=110 * 1024 * 1024),
    )(counts, post, fwt, x, W_gate_up, W_gate_up, W_down)


def _shared_body(x_ref, wg_ref, wu_ref, wd_ref, out_ref):
    f = pl.program_id(0)
    r = pl.program_id(1)
    xr = x_ref[...]
    gate = jax.lax.dot_general(
        xr, wg_ref[...], (((1,), (0,)), ((), ())),
        preferred_element_type=jnp.float32)
    up = jax.lax.dot_general(
        xr, wu_ref[...], (((1,), (0,)), ((), ())),
        preferred_element_type=jnp.float32)
    act = jax.nn.silu(gate) * up
    part = jax.lax.dot_general(
        act, wd_ref[...], (((1,), (0,)), ((), ())),
        preferred_element_type=jnp.float32)
    rows = pl.ds(r * TR, TR)

    @pl.when(f == 0)
    def _():
        out_ref[rows, :] = part

    @pl.when(f > 0)
    def _():
        out_ref[rows, :] += part


def _shared_block(x, Ws_gate_up, Ws_down):
    return pl.pallas_call(
        _shared_body,
        grid=(NF, NR),
        in_specs=[
            pl.BlockSpec((TR, D), lambda f, r: (r, 0)),
            pl.BlockSpec((D, TF), lambda f, r: (0, f)),
            pl.BlockSpec((D, TF), lambda f, r: (0, f + NF)),
            pl.BlockSpec((TF, D), lambda f, r: (f, 0)),
        ],
        out_specs=pl.BlockSpec((T, D), lambda f, r: (0, 0)),
        out_shape=jax.ShapeDtypeStruct((T, D), jnp.float32),
        compiler_params=pltpu.CompilerParams(
            dimension_semantics=("arbitrary", "arbitrary"),
            vmem_limit_bytes=100 * 1024 * 1024),
    )(x, Ws_gate_up, Ws_gate_up, Ws_down)


@jax.jit
def _run(hidden_states, Wg, W_gate_up, W_down, Ws_gate_up, Ws_down):
    orig_shape = hidden_states.shape
    x = hidden_states.reshape(-1, D)
    counts, post, fwt = _routing(x, Wg)
    counts = counts.reshape(E)
    post = post.reshape(E, 1, T)
    fwt = fwt.reshape(E, 1, T)
    moe_out = _expert_block(counts, post, fwt, x, W_gate_up, W_down)
    shared = _shared_block(x, Ws_gate_up, Ws_down)
    return (moe_out + shared).reshape(orig_shape)


def kernel(hidden_states, Wg, W_gate_up, W_down, Ws_gate_up, Ws_down):
    return _run(hidden_states, Wg, W_gate_up, W_down, Ws_gate_up, Ws_down)
